# baseline (speedup 1.0000x reference)
import jax
import jax.numpy as jnp
from jax.experimental import pallas as pl

_B, _V = 128, 100000
_BB = 8


def _softmax_body(l_ref, n_ref, o_ref):
    x = l_ref[...] + n_ref[...]
    m = jnp.max(x, axis=-1, keepdims=True)
    e = jnp.exp(x - m)
    s = jnp.sum(e, axis=-1, keepdims=True)
    o_ref[...] = e / s


def kernel(logits, gumbel_noise):
    grid = (_B // _BB,)
    spec = pl.BlockSpec((_BB, _V), lambda i: (i, 0))
    return pl.pallas_call(
        _softmax_body,
        grid=grid,
        in_specs=[spec, spec],
        out_specs=spec,
        out_shape=jax.ShapeDtypeStruct((_B, _V), jnp.float32),
    )(logits, gumbel_noise)


# speedup vs baseline: 1.0269x; 1.0269x over previous
import jax
import jax.numpy as jnp
from jax.experimental import pallas as pl

_B, _V = 128, 100000
_BB = 16


def _softmax_body(l_ref, n_ref, o_ref):
    x = l_ref[...] + n_ref[...]
    m = jnp.max(x, axis=-1, keepdims=True)
    e = jnp.exp(x - m)
    s = jnp.sum(e, axis=-1, keepdims=True)
    o_ref[...] = e / s


def kernel(logits, gumbel_noise):
    grid = (_B // _BB,)
    spec = pl.BlockSpec((_BB, _V), lambda i: (i, 0))
    return pl.pallas_call(
        _softmax_body,
        grid=grid,
        in_specs=[spec, spec],
        out_specs=spec,
        out_shape=jax.ShapeDtypeStruct((_B, _V), jnp.float32),
    )(logits, gumbel_noise)


# speedup vs baseline: 1.0810x; 1.0526x over previous
import jax
import jax.numpy as jnp
from jax.experimental import pallas as pl

_B, _V = 128, 100000
_BB = 16


def _softmax_body(x_ref, o_ref):
    x = x_ref[...]
    m = jnp.max(x, axis=-1, keepdims=True)
    e = jnp.exp(x - m)
    s = jnp.sum(e, axis=-1, keepdims=True)
    o_ref[...] = e / s


def kernel(logits, gumbel_noise):
    x = logits + gumbel_noise
    grid = (_B // _BB,)
    spec = pl.BlockSpec((_BB, _V), lambda i: (i, 0))
    return pl.pallas_call(
        _softmax_body,
        grid=grid,
        in_specs=[spec],
        out_specs=spec,
        out_shape=jax.ShapeDtypeStruct((_B, _V), jnp.float32),
    )(x)
